# Initial kernel scaffold; baseline (speedup 1.0000x reference)
import functools

import jax
import jax.numpy as jnp
from jax import lax
from jax.experimental import pallas as pl
from jax.experimental.pallas import tpu as pltpu
from jax.experimental.pallas import tpu_sc as plsc

N_NODES = 50000
N_EDGES = 800000
N_GRAPHS = 64
D_H = 64
D_E = 16
D_IN = 128
PK = 80
E_PAD = 819200
W = 128
NHALF = 25000
SP_ROWS = 25088
TRASH = 25080
ROWS_PER_SUB = 1568
ZROWS = 112
OOR = 1 << 28

_mesh = plsc.VectorSubcoreMesh(core_axis_name="c", subcore_axis_name="s")


def _sc_gather(table, idx_s, idx_d):
    n_chunks = E_PAD // W

    @functools.partial(
        pl.kernel,
        out_type=(
            jax.ShapeDtypeStruct((E_PAD, PK), jnp.float32),
            jax.ShapeDtypeStruct((E_PAD, PK), jnp.float32),
        ),
        mesh=_mesh,
    )
    def k(tab_hbm, is_hbm, id_hbm, os_hbm, od_hbm):
        def body(is_v, id_v, os_v, od_v):
            pltpu.sync_copy(tab_hbm.at[is_v], os_v)
            pltpu.sync_copy(tab_hbm.at[id_v], od_v)

        pltpu.emit_pipeline(
            body,
            grid=(n_chunks,),
            in_specs=[
                pl.BlockSpec((W,), lambda i: (i,)),
                pl.BlockSpec((W,), lambda i: (i,)),
            ],
            out_specs=[
                pl.BlockSpec((W, PK), lambda i: (i, 0)),
                pl.BlockSpec((W, PK), lambda i: (i, 0)),
            ],
            core_axis_name=("c", "s"),
            dimension_semantics=(pltpu.PARALLEL,),
        )(is_hbm, id_hbm, os_hbm, od_hbm)

    return k(table, idx_s, idx_d)


def _zero_fill(ref, rows, cols):
    z = jnp.zeros((16,), jnp.float32)

    @pl.loop(0, rows)
    def _(r):
        @pl.loop(0, cols // 16)
        def _(j):
            ref[r, pl.ds(j * 16, 16)] = z


def _localize(idx_v, out_ref, base):
    @pl.loop(0, W // 16)
    def _(j):
        sl = pl.ds(j * 16, 16)
        v = idx_v[sl] - base
        ok = (v >= 0) & (v < NHALF)
        out_ref[sl] = jnp.where(ok, v, TRASH)


def _spmem_zero(zb, sp, s):
    @pl.loop(0, ROWS_PER_SUB // ZROWS)
    def _(k_):
        pltpu.sync_copy(zb, sp.at[pl.ds(s * ROWS_PER_SUB + k_ * ZROWS, ZROWS)])


def _spmem_drain(sp, out_hbm, c, s):
    @pl.when(s < 15)
    def _():
        r0 = s * ROWS_PER_SUB
        pltpu.sync_copy(
            sp.at[pl.ds(r0, ROWS_PER_SUB)],
            out_hbm.at[pl.ds(c * NHALF + r0, ROWS_PER_SUB)],
        )

    @pl.when(s == 15)
    def _():
        r0 = 15 * ROWS_PER_SUB
        pltpu.sync_copy(
            sp.at[pl.ds(r0, NHALF - r0)],
            out_hbm.at[pl.ds(c * NHALF + r0, NHALF - r0)],
        )


def _sc_scatter(msg, wd, dst_s, src_s):
    n_chunks = E_PAD // W

    @functools.partial(
        pl.kernel,
        out_type=(
            jax.ShapeDtypeStruct((N_NODES, 64), jnp.float32),
            jax.ShapeDtypeStruct((N_NODES, 16), jnp.float32),
        ),
        mesh=_mesh,
        scratch_types=[
            pltpu.VMEM_SHARED((SP_ROWS, 64), jnp.float32),
            pltpu.VMEM_SHARED((SP_ROWS, 16), jnp.float32),
            pltpu.VMEM((ZROWS, 64), jnp.float32),
            pltpu.VMEM((ZROWS, 16), jnp.float32),
            pltpu.VMEM((W,), jnp.int32),
            pltpu.VMEM((W,), jnp.int32),
        ],
    )
    def k(msg_hbm, wd_hbm, dst_hbm, src_hbm, agg_hbm, cu_hbm,
          sp64, sp16, zb64, zb16, li_d, li_s):
        c = lax.axis_index("c")
        s = lax.axis_index("s")
        base = c * NHALF
        _zero_fill(zb64, ZROWS, 64)
        _zero_fill(zb16, ZROWS, 16)
        _spmem_zero(zb64, sp64, s)
        _spmem_zero(zb16, sp16, s)
        plsc.subcore_barrier()

        def body(dst_v, src_v, msg_v, wd_v):
            _localize(dst_v, li_d, base)
            _localize(src_v, li_s, base)
            pltpu.sync_copy(msg_v, sp64.at[li_d], add=True)
            pltpu.sync_copy(wd_v, sp16.at[li_s], add=True)

        pltpu.emit_pipeline(
            body,
            grid=(n_chunks,),
            in_specs=[
                pl.BlockSpec((W,), lambda i: (i,)),
                pl.BlockSpec((W,), lambda i: (i,)),
                pl.BlockSpec((W, 64), lambda i: (i, 0)),
                pl.BlockSpec((W, 16), lambda i: (i, 0)),
            ],
            core_axis_name="s",
            dimension_semantics=(pltpu.PARALLEL,),
        )(dst_hbm, src_hbm, msg_hbm, wd_hbm)

        plsc.subcore_barrier()
        _spmem_drain(sp64, agg_hbm, c, s)
        _spmem_drain(sp16, cu_hbm, c, s)

    return k(msg, wd, dst_s, src_s)


def _sc_counts(dst_s):
    n_chunks = E_PAD // W

    @functools.partial(
        pl.kernel,
        out_type=jax.ShapeDtypeStruct((N_NODES, 16), jnp.float32),
        mesh=_mesh,
        scratch_types=[
            pltpu.VMEM_SHARED((SP_ROWS, 16), jnp.float32),
            pltpu.VMEM((ZROWS, 16), jnp.float32),
            pltpu.VMEM((W, 16), jnp.float32),
            pltpu.VMEM((W,), jnp.int32),
        ],
    )
    def k(dst_hbm, cnt_hbm, sp16, zb16, ones_v, li):
        c = lax.axis_index("c")
        s = lax.axis_index("s")
        base = c * NHALF
        _zero_fill(zb16, ZROWS, 16)
        one_row = jnp.where(lax.iota(jnp.int32, 16) == 0, 1.0, 0.0).astype(
            jnp.float32)

        @pl.loop(0, W)
        def _(r):
            ones_v[r, pl.ds(0, 16)] = one_row

        _spmem_zero(zb16, sp16, s)
        plsc.subcore_barrier()

        def body(dst_v):
            _localize(dst_v, li, base)
            pltpu.sync_copy(ones_v, sp16.at[li], add=True)

        pltpu.emit_pipeline(
            body,
            grid=(n_chunks,),
            in_specs=[pl.BlockSpec((W,), lambda i: (i,))],
            core_axis_name="s",
            dimension_semantics=(pltpu.PARALLEL,),
        )(dst_hbm)

        plsc.subcore_barrier()
        _spmem_drain(sp16, cnt_hbm, c, s)

    return k(dst_s)


def _silu(v):
    return v * jax.nn.sigmoid(v)


BLK_E = 2048
BLK_N = 2000


def _rep(shape):
    return pl.BlockSpec(shape, lambda i: tuple(0 for _ in shape))


def _edge_mlp(gs, gd, ef, w1hs, w1hd, w1sq, w1ea, b1, w2, b2, wx1, bx1,
              wx2, bx2):
    def body(gs_r, gd_r, ef_r, w1hs_r, w1hd_r, w1sq_r, w1ea_r, b1_r, w2_r,
             b2_r, wx1_r, bx1_r, wx2_r, bx2_r, msg_r, wd_r):
        hs = gs_r[:, 0:64]
        hd = gd_r[:, 0:64]
        xs = gs_r[:, 64:67]
        xd = gd_r[:, 64:67]
        diff = xs - xd
        sq = jnp.sum(diff * diff, axis=1, keepdims=True)
        t = (jnp.dot(hs, w1hs_r[...], preferred_element_type=jnp.float32)
             + jnp.dot(hd, w1hd_r[...], preferred_element_type=jnp.float32)
             + sq * w1sq_r[...]
             + jnp.dot(ef_r[...], w1ea_r[...],
                       preferred_element_type=jnp.float32)
             + b1_r[...])
        t = _silu(t)
        msg = _silu(jnp.dot(t, w2_r[...], preferred_element_type=jnp.float32)
                    + b2_r[...])
        msg_r[...] = msg
        t3 = _silu(jnp.dot(msg, wx1_r[...],
                           preferred_element_type=jnp.float32) + bx1_r[...])
        wgt = jnp.dot(t3, wx2_r[...],
                      preferred_element_type=jnp.float32) + bx2_r[...]
        wd3 = diff * wgt
        wd_r[...] = jnp.concatenate(
            [wd3, jnp.zeros((BLK_E, 13), jnp.float32)], axis=1)

    return pl.pallas_call(
        body,
        grid=(E_PAD // BLK_E,),
        in_specs=[
            pl.BlockSpec((BLK_E, PK), lambda i: (i, 0)),
            pl.BlockSpec((BLK_E, PK), lambda i: (i, 0)),
            pl.BlockSpec((BLK_E, 16), lambda i: (i, 0)),
            _rep((64, 64)), _rep((64, 64)), _rep((1, 64)), _rep((16, 64)),
            _rep((1, 64)), _rep((64, 64)), _rep((1, 64)), _rep((64, 64)),
            _rep((1, 64)), _rep((64, 1)), _rep((1, 1)),
        ],
        out_specs=[
            pl.BlockSpec((BLK_E, 64), lambda i: (i, 0)),
            pl.BlockSpec((BLK_E, 16), lambda i: (i, 0)),
        ],
        out_shape=[
            jax.ShapeDtypeStruct((E_PAD, 64), jnp.float32),
            jax.ShapeDtypeStruct((E_PAD, 16), jnp.float32),
        ],
    )(gs, gd, ef, w1hs, w1hd, w1sq, w1ea, b1, w2, b2, wx1, bx1, wx2, bx2)


def _node_update(pk, agg, cu, cnt, wh1h, wh1a, bh1, wh2, bh2, ln_g, ln_b):
    def body(pk_r, agg_r, cu_r, cnt_r, wh1h_r, wh1a_r, bh1_r, wh2_r, bh2_r,
             g_r, b_r, out_r):
        h = pk_r[:, 0:64]
        x = pk_r[:, 64:67]
        rc = 1.0 / jnp.maximum(cnt_r[:, 0:1], 1.0)
        agg_n = agg_r[...] * rc
        t = _silu(jnp.dot(h, wh1h_r[...], preferred_element_type=jnp.float32)
                  + jnp.dot(agg_n, wh1a_r[...],
                            preferred_element_type=jnp.float32)
                  + bh1_r[...])
        hh = jnp.dot(t, wh2_r[...],
                     preferred_element_type=jnp.float32) + bh2_r[...]
        pre = h + hh
        mu = jnp.mean(pre, axis=1, keepdims=True)
        d = pre - mu
        var = jnp.mean(d * d, axis=1, keepdims=True)
        hn = d * lax.rsqrt(var + 1e-5) * g_r[...] + b_r[...]
        xn = x + cu_r[:, 0:3] * rc
        out_r[...] = jnp.concatenate(
            [hn, xn, jnp.zeros((BLK_N, 13), jnp.float32)], axis=1)

    return pl.pallas_call(
        body,
        grid=(N_NODES // BLK_N,),
        in_specs=[
            pl.BlockSpec((BLK_N, PK), lambda i: (i, 0)),
            pl.BlockSpec((BLK_N, 64), lambda i: (i, 0)),
            pl.BlockSpec((BLK_N, 16), lambda i: (i, 0)),
            pl.BlockSpec((BLK_N, 16), lambda i: (i, 0)),
            _rep((64, 64)), _rep((64, 64)), _rep((1, 64)),
            _rep((64, 64)), _rep((1, 64)), _rep((1, 64)), _rep((1, 64)),
        ],
        out_specs=pl.BlockSpec((BLK_N, PK), lambda i: (i, 0)),
        out_shape=jax.ShapeDtypeStruct((N_NODES, PK), jnp.float32),
    )(pk, agg, cu, cnt, wh1h, wh1a, bh1, wh2, bh2, ln_g, ln_b)


def _encoder(nf, coords, w0, b0, w1, b1):
    def body(nf_r, x_r, w0_r, b0_r, w1_r, b1_r, out_r):
        t = _silu(jnp.dot(nf_r[...], w0_r[...],
                          preferred_element_type=jnp.float32) + b0_r[...])
        h = jnp.dot(t, w1_r[...],
                    preferred_element_type=jnp.float32) + b1_r[...]
        out_r[...] = jnp.concatenate(
            [h, x_r[...], jnp.zeros((BLK_N, 13), jnp.float32)], axis=1)

    return pl.pallas_call(
        body,
        grid=(N_NODES // BLK_N,),
        in_specs=[
            pl.BlockSpec((BLK_N, D_IN), lambda i: (i, 0)),
            pl.BlockSpec((BLK_N, 3), lambda i: (i, 0)),
            _rep((D_IN, 64)), _rep((1, 64)), _rep((64, 64)), _rep((1, 64)),
        ],
        out_specs=pl.BlockSpec((BLK_N, PK), lambda i: (i, 0)),
        out_shape=jax.ShapeDtypeStruct((N_NODES, PK), jnp.float32),
    )(nf, coords, w0, b0, w1, b1)


def _readout(pk, batch2, r0, br0, r1, br1, r2, br2):
    n_steps = N_NODES // BLK_N

    def body(pk_r, b_r, r0_r, br0_r, r1_r, br1_r, r2_r, br2_r, out_r,
             gh_acc, ct_acc):
        i = pl.program_id(0)

        @pl.when(i == 0)
        def _():
            gh_acc[...] = jnp.zeros((N_GRAPHS, 64), jnp.float32)
            ct_acc[...] = jnp.zeros((N_GRAPHS, 1), jnp.float32)

        h = pk_r[:, 0:64]
        gid = jax.lax.broadcasted_iota(jnp.int32, (BLK_N, N_GRAPHS), 1)
        z = (b_r[...] == gid).astype(jnp.float32)
        gh_acc[...] += lax.dot_general(
            z, h, (((0,), (0,)), ((), ())),
            preferred_element_type=jnp.float32)
        ct_acc[...] += lax.dot_general(
            z, jnp.ones((BLK_N, 1), jnp.float32), (((0,), (0,)), ((), ())),
            preferred_element_type=jnp.float32)

        @pl.when(i == n_steps - 1)
        def _():
            gm = gh_acc[...] / jnp.maximum(ct_acc[...], 1.0)
            g0 = _silu(jnp.dot(gm, r0_r[...],
                               preferred_element_type=jnp.float32)
                       + br0_r[...])
            g1 = _silu(jnp.dot(g0, r1_r[...],
                               preferred_element_type=jnp.float32)
                       + br1_r[...])
            out_r[...] = jnp.dot(
                g1, r2_r[...], preferred_element_type=jnp.float32) + br2_r[...]

    return pl.pallas_call(
        body,
        grid=(n_steps,),
        in_specs=[
            pl.BlockSpec((BLK_N, PK), lambda i: (i, 0)),
            pl.BlockSpec((BLK_N, 1), lambda i: (i, 0)),
            _rep((64, 64)), _rep((1, 64)), _rep((64, 32)), _rep((1, 32)),
            _rep((32, 2)), _rep((1, 2)),
        ],
        out_specs=pl.BlockSpec((N_GRAPHS, 2), lambda i: (0, 0)),
        out_shape=jax.ShapeDtypeStruct((N_GRAPHS, 2), jnp.float32),
        scratch_shapes=[
            pltpu.VMEM((N_GRAPHS, 64), jnp.float32),
            pltpu.VMEM((N_GRAPHS, 1), jnp.float32),
        ],
    )(pk, batch2, r0, br0, r1, br1, r2, br2)


def _row(b):
    return b.reshape(1, -1)


def kernel(node_feats, coords, edge_index, edge_feats, batch, params):
    f32 = jnp.float32
    src = edge_index[0].astype(jnp.int32)
    dst = edge_index[1].astype(jnp.int32)
    npad = E_PAD - N_EDGES
    zpad_i = jnp.zeros((npad,), jnp.int32)
    oor = jnp.full((npad,), OOR, jnp.int32)
    src_g = jnp.concatenate([src, zpad_i])
    dst_g = jnp.concatenate([dst, zpad_i])
    src_s = jnp.concatenate([src, oor])
    dst_s = jnp.concatenate([dst, oor])
    ef_p = jnp.concatenate(
        [edge_feats.astype(f32), jnp.zeros((npad, D_E), f32)], axis=0)
    batch2 = batch.astype(jnp.int32).reshape(N_NODES, 1)

    enc0, enc1 = params["enc"]
    table = _encoder(node_feats.astype(f32), coords.astype(f32),
                     enc0["W"].T, _row(enc0["b"]), enc1["W"].T,
                     _row(enc1["b"]))

    cnt = _sc_counts(dst_s)

    for p in params["layers"]:
        w1 = p["e1"]["W"]
        gs, gd = _sc_gather(table, src_g, dst_g)
        msg, wd = _edge_mlp(
            gs, gd, ef_p,
            w1[:, 0:64].T, w1[:, 64:128].T, _row(w1[:, 128]),
            w1[:, 129:145].T, _row(p["e1"]["b"]),
            p["e2"]["W"].T, _row(p["e2"]["b"]),
            p["x1"]["W"].T, _row(p["x1"]["b"]),
            p["x2"]["W"].T, _row(p["x2"]["b"]),
        )
        agg, cu = _sc_scatter(msg, wd, dst_s, src_s)
        wh1 = p["h1"]["W"]
        table = _node_update(
            table, agg, cu, cnt,
            wh1[:, 0:64].T, wh1[:, 64:128].T, _row(p["h1"]["b"]),
            p["h2"]["W"].T, _row(p["h2"]["b"]),
            _row(p["ln_g"]), _row(p["ln_b"]),
        )

    r = params["ro"]
    return _readout(table, batch2, r[0]["W"].T, _row(r[0]["b"]),
                    r[1]["W"].T, _row(r[1]["b"]), r[2]["W"].T,
                    _row(r[2]["b"]))


# speedup vs baseline: 18.2759x; 18.2759x over previous
import functools

import jax
import jax.numpy as jnp
from jax import lax
from jax.experimental import pallas as pl
from jax.experimental.pallas import tpu as pltpu
from jax.experimental.pallas import tpu_sc as plsc

N_NODES = 50000
N_EDGES = 800000
N_GRAPHS = 64
D_H = 64
D_E = 16
D_IN = 128
XW = 16
E_PAD = 819200
W = 128
NHALF = 25000
SP_ROWS = 25088
TRASH = 25080
ROWS_PER_SUB = 1568
ZROWS = 112
OOR = 1 << 28

_mesh = plsc.VectorSubcoreMesh(core_axis_name="c", subcore_axis_name="s")
_sc_params = pltpu.CompilerParams(use_tc_tiling_on_sc=False)


def _sc_gather(htab, xtab, idx_s, idx_d):
    n_chunks = E_PAD // W

    @functools.partial(
        pl.kernel,
        out_type=(
            jax.ShapeDtypeStruct((E_PAD, D_H), jnp.bfloat16),
            jax.ShapeDtypeStruct((E_PAD, D_H), jnp.bfloat16),
            jax.ShapeDtypeStruct((E_PAD, XW), jnp.float32),
            jax.ShapeDtypeStruct((E_PAD, XW), jnp.float32),
        ),
        mesh=_mesh,
        compiler_params=_sc_params,
    )
    def k(h_hbm, x_hbm, is_hbm, id_hbm, hs_hbm, hd_hbm, xs_hbm, xd_hbm):
        def body(is_v, id_v, hs_v, hd_v, xs_v, xd_v):
            pltpu.sync_copy(h_hbm.at[is_v], hs_v)
            pltpu.sync_copy(h_hbm.at[id_v], hd_v)
            pltpu.sync_copy(x_hbm.at[is_v], xs_v)
            pltpu.sync_copy(x_hbm.at[id_v], xd_v)

        pltpu.emit_pipeline(
            body,
            grid=(n_chunks,),
            in_specs=[
                pl.BlockSpec((W,), lambda i: (i,)),
                pl.BlockSpec((W,), lambda i: (i,)),
            ],
            out_specs=[
                pl.BlockSpec((W, D_H), lambda i: (i, 0)),
                pl.BlockSpec((W, D_H), lambda i: (i, 0)),
                pl.BlockSpec((W, XW), lambda i: (i, 0)),
                pl.BlockSpec((W, XW), lambda i: (i, 0)),
            ],
            core_axis_name=("c", "s"),
            dimension_semantics=(pltpu.PARALLEL,),
        )(is_hbm, id_hbm, hs_hbm, hd_hbm, xs_hbm, xd_hbm)

    return k(htab, xtab, idx_s, idx_d)


def _zero_fill(ref, rows, cols):
    z = jnp.zeros((16,), jnp.float32)

    @pl.loop(0, rows)
    def _(r):
        @pl.loop(0, cols // 16)
        def _(j):
            ref[r, pl.ds(j * 16, 16)] = z


def _localize(idx_v, out_ref, base):
    @pl.loop(0, W // 16)
    def _(j):
        sl = pl.ds(j * 16, 16)
        v = idx_v[sl] - base
        ok = (v >= 0) & (v < NHALF)
        out_ref[sl] = jnp.where(ok, v, TRASH)


def _spmem_zero(zb, sp, s):
    @pl.loop(0, ROWS_PER_SUB // ZROWS)
    def _(k_):
        pltpu.sync_copy(zb, sp.at[pl.ds(s * ROWS_PER_SUB + k_ * ZROWS, ZROWS)])


def _spmem_drain(sp, out_hbm, c, s):
    @pl.when(s < 15)
    def _():
        r0 = s * ROWS_PER_SUB
        pltpu.sync_copy(
            sp.at[pl.ds(r0, ROWS_PER_SUB)],
            out_hbm.at[pl.ds(c * NHALF + r0, ROWS_PER_SUB)],
        )

    @pl.when(s == 15)
    def _():
        r0 = 15 * ROWS_PER_SUB
        pltpu.sync_copy(
            sp.at[pl.ds(r0, NHALF - r0)],
            out_hbm.at[pl.ds(c * NHALF + r0, NHALF - r0)],
        )


def _sc_scatter(vals, idx_s, ncols):
    n_chunks = E_PAD // W

    @functools.partial(
        pl.kernel,
        out_type=jax.ShapeDtypeStruct((N_NODES, ncols), jnp.float32),
        mesh=_mesh,
        scratch_types=[
            pltpu.VMEM_SHARED((SP_ROWS, ncols), jnp.float32),
            pltpu.VMEM((ZROWS, ncols), jnp.float32),
            pltpu.VMEM((W,), jnp.int32),
        ],
        compiler_params=_sc_params,
    )
    def k(vals_hbm, idx_hbm, out_hbm, sp, zb, li):
        c = lax.axis_index("c")
        s = lax.axis_index("s")
        base = c * NHALF
        _zero_fill(zb, ZROWS, ncols)
        _spmem_zero(zb, sp, s)
        plsc.subcore_barrier()

        def body(idx_v, vals_v):
            _localize(idx_v, li, base)
            pltpu.sync_copy(vals_v, sp.at[li], add=True)

        pltpu.emit_pipeline(
            body,
            grid=(n_chunks,),
            in_specs=[
                pl.BlockSpec((W,), lambda i: (i,)),
                pl.BlockSpec((W, ncols), lambda i: (i, 0)),
            ],
            core_axis_name="s",
            dimension_semantics=(pltpu.PARALLEL,),
        )(idx_hbm, vals_hbm)

        plsc.subcore_barrier()
        _spmem_drain(sp, out_hbm, c, s)

    return k(vals, idx_s)


def _sc_counts(dst_s):
    n_chunks = E_PAD // W

    @functools.partial(
        pl.kernel,
        out_type=jax.ShapeDtypeStruct((N_NODES, 16), jnp.float32),
        mesh=_mesh,
        scratch_types=[
            pltpu.VMEM_SHARED((SP_ROWS, 16), jnp.float32),
            pltpu.VMEM((ZROWS, 16), jnp.float32),
            pltpu.VMEM((W, 16), jnp.float32),
            pltpu.VMEM((W,), jnp.int32),
        ],
        compiler_params=_sc_params,
    )
    def k(dst_hbm, cnt_hbm, sp16, zb16, ones_v, li):
        c = lax.axis_index("c")
        s = lax.axis_index("s")
        base = c * NHALF
        _zero_fill(zb16, ZROWS, 16)
        one_row = jnp.where(lax.iota(jnp.int32, 16) == 0, 1.0, 0.0).astype(
            jnp.float32)

        @pl.loop(0, W)
        def _(r):
            ones_v[r, pl.ds(0, 16)] = one_row

        _spmem_zero(zb16, sp16, s)
        plsc.subcore_barrier()

        def body(dst_v):
            _localize(dst_v, li, base)
            pltpu.sync_copy(ones_v, sp16.at[li], add=True)

        pltpu.emit_pipeline(
            body,
            grid=(n_chunks,),
            in_specs=[pl.BlockSpec((W,), lambda i: (i,))],
            core_axis_name="s",
            dimension_semantics=(pltpu.PARALLEL,),
        )(dst_hbm)

        plsc.subcore_barrier()
        _spmem_drain(sp16, cnt_hbm, c, s)

    return k(dst_s)


def _silu(v):
    return v * jax.nn.sigmoid(v)


BLK_E = 2048
BLK_N = 2000
bf16 = jnp.bfloat16


def _rep(shape):
    return pl.BlockSpec(shape, lambda i: tuple(0 for _ in shape))


def _bdot(a, b):
    return jnp.dot(a, b, preferred_element_type=jnp.float32)


def _edge_mlp(hs, hd, xs, xd, ef, w1hs, w1hd, w1sq, w1ea, b1, w2, b2, wx1,
              bx1, wx2, bx2):
    def body(hs_r, hd_r, xs_r, xd_r, ef_r, w1hs_r, w1hd_r, w1sq_r, w1ea_r,
             b1_r, w2_r, b2_r, wx1_r, bx1_r, wx2_r, bx2_r, msg_r, wd_r):
        xsv = xs_r[:, 0:3]
        xdv = xd_r[:, 0:3]
        diff = xsv - xdv
        sq = jnp.sum(diff * diff, axis=1, keepdims=True)
        t = (_bdot(hs_r[...], w1hs_r[...])
             + _bdot(hd_r[...], w1hd_r[...])
             + sq * w1sq_r[...]
             + _bdot(ef_r[...], w1ea_r[...])
             + b1_r[...])
        t = _silu(t)
        msg = _silu(_bdot(t.astype(bf16), w2_r[...]) + b2_r[...])
        msg_r[...] = msg
        mb = msg.astype(bf16)
        t3 = _silu(_bdot(mb, wx1_r[...]) + bx1_r[...])
        wgt = _bdot(t3.astype(bf16), wx2_r[...]) + bx2_r[...]
        wd3 = diff * wgt
        wd_r[...] = jnp.concatenate(
            [wd3, jnp.zeros((BLK_E, 13), jnp.float32)], axis=1)

    return pl.pallas_call(
        body,
        grid=(E_PAD // BLK_E,),
        in_specs=[
            pl.BlockSpec((BLK_E, D_H), lambda i: (i, 0)),
            pl.BlockSpec((BLK_E, D_H), lambda i: (i, 0)),
            pl.BlockSpec((BLK_E, XW), lambda i: (i, 0)),
            pl.BlockSpec((BLK_E, XW), lambda i: (i, 0)),
            pl.BlockSpec((BLK_E, 16), lambda i: (i, 0)),
            _rep((64, 64)), _rep((64, 64)), _rep((1, 64)), _rep((16, 64)),
            _rep((1, 64)), _rep((64, 64)), _rep((1, 64)), _rep((64, 64)),
            _rep((1, 64)), _rep((64, 1)), _rep((1, 1)),
        ],
        out_specs=[
            pl.BlockSpec((BLK_E, 64), lambda i: (i, 0)),
            pl.BlockSpec((BLK_E, 16), lambda i: (i, 0)),
        ],
        out_shape=[
            jax.ShapeDtypeStruct((E_PAD, 64), jnp.float32),
            jax.ShapeDtypeStruct((E_PAD, 16), jnp.float32),
        ],
    )(hs, hd, xs, xd, ef, w1hs, w1hd, w1sq, w1ea, b1, w2, b2, wx1, bx1,
      wx2, bx2)


def _node_update(h, xt, agg, cu, cnt, wh1h, wh1a, bh1, wh2, bh2, ln_g, ln_b):
    def body(h_r, xt_r, agg_r, cu_r, cnt_r, wh1h_r, wh1a_r, bh1_r, wh2_r,
             bh2_r, g_r, b_r, hn_r, hb_r, xn_r):
        h_ = h_r[...]
        x = xt_r[:, 0:3]
        rc = 1.0 / jnp.maximum(cnt_r[:, 0:1], 1.0)
        agg_n = agg_r[...] * rc
        t = _silu(_bdot(h_, wh1h_r[...]) + _bdot(agg_n, wh1a_r[...])
                  + bh1_r[...])
        hh = _bdot(t, wh2_r[...]) + bh2_r[...]
        pre = h_ + hh
        mu = jnp.mean(pre, axis=1, keepdims=True)
        d = pre - mu
        var = jnp.mean(d * d, axis=1, keepdims=True)
        hn = d * lax.rsqrt(var + 1e-5) * g_r[...] + b_r[...]
        xn = x + cu_r[:, 0:3] * rc
        hn_r[...] = hn
        hb_r[...] = hn.astype(bf16)
        xn_r[...] = jnp.concatenate(
            [xn, jnp.zeros((BLK_N, XW - 3), jnp.float32)], axis=1)

    return pl.pallas_call(
        body,
        grid=(N_NODES // BLK_N,),
        in_specs=[
            pl.BlockSpec((BLK_N, D_H), lambda i: (i, 0)),
            pl.BlockSpec((BLK_N, XW), lambda i: (i, 0)),
            pl.BlockSpec((BLK_N, 64), lambda i: (i, 0)),
            pl.BlockSpec((BLK_N, 16), lambda i: (i, 0)),
            pl.BlockSpec((BLK_N, 16), lambda i: (i, 0)),
            _rep((64, 64)), _rep((64, 64)), _rep((1, 64)),
            _rep((64, 64)), _rep((1, 64)), _rep((1, 64)), _rep((1, 64)),
        ],
        out_specs=[
            pl.BlockSpec((BLK_N, D_H), lambda i: (i, 0)),
            pl.BlockSpec((BLK_N, D_H), lambda i: (i, 0)),
            pl.BlockSpec((BLK_N, XW), lambda i: (i, 0)),
        ],
        out_shape=[
            jax.ShapeDtypeStruct((N_NODES, D_H), jnp.float32),
            jax.ShapeDtypeStruct((N_NODES, D_H), jnp.bfloat16),
            jax.ShapeDtypeStruct((N_NODES, XW), jnp.float32),
        ],
    )(h, xt, agg, cu, cnt, wh1h, wh1a, bh1, wh2, bh2, ln_g, ln_b)


def _encoder(nf, coords, w0, b0, w1, b1):
    def body(nf_r, x_r, w0_r, b0_r, w1_r, b1_r, hn_r, hb_r, xn_r):
        t = _silu(_bdot(nf_r[...], w0_r[...]) + b0_r[...])
        h = _bdot(t, w1_r[...]) + b1_r[...]
        hn_r[...] = h
        hb_r[...] = h.astype(bf16)
        xn_r[...] = jnp.concatenate(
            [x_r[...], jnp.zeros((BLK_N, XW - 3), jnp.float32)], axis=1)

    return pl.pallas_call(
        body,
        grid=(N_NODES // BLK_N,),
        in_specs=[
            pl.BlockSpec((BLK_N, D_IN), lambda i: (i, 0)),
            pl.BlockSpec((BLK_N, 3), lambda i: (i, 0)),
            _rep((D_IN, 64)), _rep((1, 64)), _rep((64, 64)), _rep((1, 64)),
        ],
        out_specs=[
            pl.BlockSpec((BLK_N, D_H), lambda i: (i, 0)),
            pl.BlockSpec((BLK_N, D_H), lambda i: (i, 0)),
            pl.BlockSpec((BLK_N, XW), lambda i: (i, 0)),
        ],
        out_shape=[
            jax.ShapeDtypeStruct((N_NODES, D_H), jnp.float32),
            jax.ShapeDtypeStruct((N_NODES, D_H), jnp.bfloat16),
            jax.ShapeDtypeStruct((N_NODES, XW), jnp.float32),
        ],
    )(nf, coords, w0, b0, w1, b1)


def _readout(h, batch2, r0, br0, r1, br1, r2, br2):
    n_steps = N_NODES // BLK_N

    def body(h_r, b_r, r0_r, br0_r, r1_r, br1_r, r2_r, br2_r, out_r,
             gh_acc, ct_acc):
        i = pl.program_id(0)

        @pl.when(i == 0)
        def _():
            gh_acc[...] = jnp.zeros((N_GRAPHS, 64), jnp.float32)
            ct_acc[...] = jnp.zeros((N_GRAPHS, 1), jnp.float32)

        h_ = h_r[...]
        gid = jax.lax.broadcasted_iota(jnp.int32, (BLK_N, N_GRAPHS), 1)
        z = (b_r[...] == gid).astype(jnp.float32)
        gh_acc[...] += lax.dot_general(
            z, h_, (((0,), (0,)), ((), ())),
            preferred_element_type=jnp.float32)
        ct_acc[...] += lax.dot_general(
            z, jnp.ones((BLK_N, 1), jnp.float32), (((0,), (0,)), ((), ())),
            preferred_element_type=jnp.float32)

        @pl.when(i == n_steps - 1)
        def _():
            gm = gh_acc[...] / jnp.maximum(ct_acc[...], 1.0)
            g0 = _silu(_bdot(gm, r0_r[...]) + br0_r[...])
            g1 = _silu(_bdot(g0, r1_r[...]) + br1_r[...])
            out_r[...] = _bdot(g1, r2_r[...]) + br2_r[...]

    return pl.pallas_call(
        body,
        grid=(n_steps,),
        in_specs=[
            pl.BlockSpec((BLK_N, D_H), lambda i: (i, 0)),
            pl.BlockSpec((BLK_N, 1), lambda i: (i, 0)),
            _rep((64, 64)), _rep((1, 64)), _rep((64, 32)), _rep((1, 32)),
            _rep((32, 2)), _rep((1, 2)),
        ],
        out_specs=pl.BlockSpec((N_GRAPHS, 2), lambda i: (0, 0)),
        out_shape=jax.ShapeDtypeStruct((N_GRAPHS, 2), jnp.float32),
        scratch_shapes=[
            pltpu.VMEM((N_GRAPHS, 64), jnp.float32),
            pltpu.VMEM((N_GRAPHS, 1), jnp.float32),
        ],
    )(h, batch2, r0, br0, r1, br1, r2, br2)


def _row(b):
    return b.reshape(1, -1)


def kernel(node_feats, coords, edge_index, edge_feats, batch, params):
    f32 = jnp.float32
    src = edge_index[0].astype(jnp.int32)
    dst = edge_index[1].astype(jnp.int32)
    npad = E_PAD - N_EDGES
    zpad_i = jnp.zeros((npad,), jnp.int32)
    oor = jnp.full((npad,), OOR, jnp.int32)
    src_g = jnp.concatenate([src, zpad_i])
    dst_g = jnp.concatenate([dst, zpad_i])
    src_s = jnp.concatenate([src, oor])
    dst_s = jnp.concatenate([dst, oor])
    ef_p = jnp.concatenate(
        [edge_feats.astype(bf16), jnp.zeros((npad, D_E), bf16)], axis=0)
    batch2 = batch.astype(jnp.int32).reshape(N_NODES, 1)

    enc0, enc1 = params["enc"]
    h, hb, xt = _encoder(node_feats.astype(f32), coords.astype(f32),
                         enc0["W"].T, _row(enc0["b"]), enc1["W"].T,
                         _row(enc1["b"]))

    cnt = _sc_counts(dst_s)

    for p in params["layers"]:
        w1 = p["e1"]["W"]
        hs, hd, xs, xd = _sc_gather(hb, xt, src_g, dst_g)
        msg, wd = _edge_mlp(
            hs, hd, xs, xd, ef_p,
            w1[:, 0:64].T.astype(bf16), w1[:, 64:128].T.astype(bf16),
            _row(w1[:, 128]), w1[:, 129:145].T.astype(bf16),
            _row(p["e1"]["b"]),
            p["e2"]["W"].T.astype(bf16), _row(p["e2"]["b"]),
            p["x1"]["W"].T.astype(bf16), _row(p["x1"]["b"]),
            p["x2"]["W"].T.astype(bf16), _row(p["x2"]["b"]),
        )
        agg = _sc_scatter(msg, dst_s, 64)
        cu = _sc_scatter(wd, src_s, 16)
        wh1 = p["h1"]["W"]
        h, hb, xt = _node_update(
            h, xt, agg, cu, cnt,
            wh1[:, 0:64].T, wh1[:, 64:128].T, _row(p["h1"]["b"]),
            p["h2"]["W"].T, _row(p["h2"]["b"]),
            _row(p["ln_g"]), _row(p["ln_b"]),
        )

    r = params["ro"]
    return _readout(h, batch2, r[0]["W"].T, _row(r[0]["b"]),
                    r[1]["W"].T, _row(r[1]["b"]), r[2]["W"].T,
                    _row(r[2]["b"]))


# speedup vs baseline: 19.6388x; 1.0746x over previous
import functools

import jax
import jax.numpy as jnp
from jax import lax
from jax.experimental import pallas as pl
from jax.experimental.pallas import tpu as pltpu
from jax.experimental.pallas import tpu_sc as plsc

N_NODES = 50000
N_EDGES = 800000
N_GRAPHS = 64
D_H = 64
D_E = 16
D_IN = 128
XW = 16
E_PAD = 819200
W = 128
NHALF = 25000
SP_ROWS = 25088
TRASH = 25080
ROWS_PER_SUB = 1568
ZROWS = 112
OOR = 1 << 28

_mesh = plsc.VectorSubcoreMesh(core_axis_name="c", subcore_axis_name="s")
_sc_params = pltpu.CompilerParams(use_tc_tiling_on_sc=False)


PKW = 128


def _sc_gather(table, idx_s, idx_d):
    n_chunks = E_PAD // W

    @functools.partial(
        pl.kernel,
        out_type=(
            jax.ShapeDtypeStruct((E_PAD, PKW), jnp.float32),
            jax.ShapeDtypeStruct((E_PAD, PKW), jnp.float32),
        ),
        mesh=_mesh,
        scratch_types=[pltpu.SemaphoreType.DMA, pltpu.SemaphoreType.DMA],
    )
    def k(tab_hbm, is_hbm, id_hbm, os_hbm, od_hbm, sem1, sem2):
        def body(is_v, id_v, os_v, od_v):
            d1 = pltpu.async_copy(tab_hbm.at[is_v], os_v, sem1)
            d2 = pltpu.async_copy(tab_hbm.at[id_v], od_v, sem2)
            d1.wait()
            d2.wait()

        pltpu.emit_pipeline(
            body,
            grid=(n_chunks,),
            in_specs=[
                pl.BlockSpec((W,), lambda i: (i,)),
                pl.BlockSpec((W,), lambda i: (i,)),
            ],
            out_specs=[
                pl.BlockSpec((W, PKW), lambda i: (i, 0)),
                pl.BlockSpec((W, PKW), lambda i: (i, 0)),
            ],
            core_axis_name=("c", "s"),
            dimension_semantics=(pltpu.PARALLEL,),
        )(is_hbm, id_hbm, os_hbm, od_hbm)

    return k(table, idx_s, idx_d)


def _zero_fill(ref, rows, cols):
    z = jnp.zeros((16,), jnp.float32)

    @pl.loop(0, rows)
    def _(r):
        @pl.loop(0, cols // 16)
        def _(j):
            ref[r, pl.ds(j * 16, 16)] = z


def _localize(idx_v, out_ref, base):
    @pl.loop(0, W // 16)
    def _(j):
        sl = pl.ds(j * 16, 16)
        v = idx_v[sl] - base
        ok = (v >= 0) & (v < NHALF)
        out_ref[sl] = jnp.where(ok, v, TRASH)


def _spmem_zero(zb, sp, s):
    @pl.loop(0, ROWS_PER_SUB // ZROWS)
    def _(k_):
        pltpu.sync_copy(zb, sp.at[pl.ds(s * ROWS_PER_SUB + k_ * ZROWS, ZROWS)])


def _spmem_drain(sp, out_hbm, c, s):
    @pl.when(s < 15)
    def _():
        r0 = s * ROWS_PER_SUB
        pltpu.sync_copy(
            sp.at[pl.ds(r0, ROWS_PER_SUB)],
            out_hbm.at[pl.ds(c * NHALF + r0, ROWS_PER_SUB)],
        )

    @pl.when(s == 15)
    def _():
        r0 = 15 * ROWS_PER_SUB
        pltpu.sync_copy(
            sp.at[pl.ds(r0, NHALF - r0)],
            out_hbm.at[pl.ds(c * NHALF + r0, NHALF - r0)],
        )


def _sc_scatter(vals, idx_s, ncols):
    n_chunks = E_PAD // W

    @functools.partial(
        pl.kernel,
        out_type=jax.ShapeDtypeStruct((N_NODES, ncols), jnp.float32),
        mesh=_mesh,
        scratch_types=[
            pltpu.VMEM_SHARED((SP_ROWS, ncols), jnp.float32),
            pltpu.VMEM((ZROWS, ncols), jnp.float32),
            pltpu.VMEM((W,), jnp.int32),
        ],
        compiler_params=_sc_params,
    )
    def k(vals_hbm, idx_hbm, out_hbm, sp, zb, li):
        c = lax.axis_index("c")
        s = lax.axis_index("s")
        base = c * NHALF
        _zero_fill(zb, ZROWS, ncols)
        _spmem_zero(zb, sp, s)
        plsc.subcore_barrier()

        def body(idx_v, vals_v):
            _localize(idx_v, li, base)
            pltpu.sync_copy(vals_v, sp.at[li], add=True)

        pltpu.emit_pipeline(
            body,
            grid=(n_chunks,),
            in_specs=[
                pl.BlockSpec((W,), lambda i: (i,)),
                pl.BlockSpec((W, ncols), lambda i: (i, 0)),
            ],
            core_axis_name="s",
            dimension_semantics=(pltpu.PARALLEL,),
        )(idx_hbm, vals_hbm)

        plsc.subcore_barrier()
        _spmem_drain(sp, out_hbm, c, s)

    return k(vals, idx_s)


def _sc_counts(dst_s):
    n_chunks = E_PAD // W

    @functools.partial(
        pl.kernel,
        out_type=jax.ShapeDtypeStruct((N_NODES, 16), jnp.float32),
        mesh=_mesh,
        scratch_types=[
            pltpu.VMEM_SHARED((SP_ROWS, 16), jnp.float32),
            pltpu.VMEM((ZROWS, 16), jnp.float32),
            pltpu.VMEM((W, 16), jnp.float32),
            pltpu.VMEM((W,), jnp.int32),
        ],
        compiler_params=_sc_params,
    )
    def k(dst_hbm, cnt_hbm, sp16, zb16, ones_v, li):
        c = lax.axis_index("c")
        s = lax.axis_index("s")
        base = c * NHALF
        _zero_fill(zb16, ZROWS, 16)
        one_row = jnp.where(lax.iota(jnp.int32, 16) == 0, 1.0, 0.0).astype(
            jnp.float32)

        @pl.loop(0, W)
        def _(r):
            ones_v[r, pl.ds(0, 16)] = one_row

        _spmem_zero(zb16, sp16, s)
        plsc.subcore_barrier()

        def body(dst_v):
            _localize(dst_v, li, base)
            pltpu.sync_copy(ones_v, sp16.at[li], add=True)

        pltpu.emit_pipeline(
            body,
            grid=(n_chunks,),
            in_specs=[pl.BlockSpec((W,), lambda i: (i,))],
            core_axis_name="s",
            dimension_semantics=(pltpu.PARALLEL,),
        )(dst_hbm)

        plsc.subcore_barrier()
        _spmem_drain(sp16, cnt_hbm, c, s)

    return k(dst_s)


def _silu(v):
    return v * jax.nn.sigmoid(v)


BLK_E = 2048
BLK_N = 2000
bf16 = jnp.bfloat16


def _rep(shape):
    return pl.BlockSpec(shape, lambda i: tuple(0 for _ in shape))


def _bdot(a, b):
    return jnp.dot(a, b, preferred_element_type=jnp.float32)


def _edge_mlp(gs, gd, ef, w1hs, w1hd, w1sq, w1ea, b1, w2, b2, wx1,
              bx1, wx2, bx2):
    def body(gs_r, gd_r, ef_r, w1hs_r, w1hd_r, w1sq_r, w1ea_r,
             b1_r, w2_r, b2_r, wx1_r, bx1_r, wx2_r, bx2_r, msg_r, wd_r):
        hs = gs_r[:, 0:64].astype(bf16)
        hd = gd_r[:, 0:64].astype(bf16)
        xsv = gs_r[:, 64:67]
        xdv = gd_r[:, 64:67]
        diff = xsv - xdv
        sq = jnp.sum(diff * diff, axis=1, keepdims=True)
        t = (_bdot(hs, w1hs_r[...])
             + _bdot(hd, w1hd_r[...])
             + sq * w1sq_r[...]
             + _bdot(ef_r[...], w1ea_r[...])
             + b1_r[...])
        t = _silu(t)
        msg = _silu(_bdot(t.astype(bf16), w2_r[...]) + b2_r[...])
        msg_r[...] = msg
        mb = msg.astype(bf16)
        t3 = _silu(_bdot(mb, wx1_r[...]) + bx1_r[...])
        wgt = _bdot(t3.astype(bf16), wx2_r[...]) + bx2_r[...]
        wd3 = diff * wgt
        wd_r[...] = jnp.concatenate(
            [wd3, jnp.zeros((BLK_E, 13), jnp.float32)], axis=1)

    return pl.pallas_call(
        body,
        grid=(E_PAD // BLK_E,),
        in_specs=[
            pl.BlockSpec((BLK_E, PKW), lambda i: (i, 0)),
            pl.BlockSpec((BLK_E, PKW), lambda i: (i, 0)),
            pl.BlockSpec((BLK_E, 16), lambda i: (i, 0)),
            _rep((64, 64)), _rep((64, 64)), _rep((1, 64)), _rep((16, 64)),
            _rep((1, 64)), _rep((64, 64)), _rep((1, 64)), _rep((64, 64)),
            _rep((1, 64)), _rep((64, 1)), _rep((1, 1)),
        ],
        out_specs=[
            pl.BlockSpec((BLK_E, 64), lambda i: (i, 0)),
            pl.BlockSpec((BLK_E, 16), lambda i: (i, 0)),
        ],
        out_shape=[
            jax.ShapeDtypeStruct((E_PAD, 64), jnp.float32),
            jax.ShapeDtypeStruct((E_PAD, 16), jnp.float32),
        ],
    )(gs, gd, ef, w1hs, w1hd, w1sq, w1ea, b1, w2, b2, wx1, bx1,
      wx2, bx2)


def _node_update(tab, agg, cu, cnt, wh1h, wh1a, bh1, wh2, bh2, ln_g, ln_b):
    def body(tab_r, agg_r, cu_r, cnt_r, wh1h_r, wh1a_r, bh1_r, wh2_r,
             bh2_r, g_r, b_r, out_r):
        h_ = tab_r[:, 0:64]
        x = tab_r[:, 64:67]
        rc = 1.0 / jnp.maximum(cnt_r[:, 0:1], 1.0)
        agg_n = agg_r[...] * rc
        t = _silu(_bdot(h_, wh1h_r[...]) + _bdot(agg_n, wh1a_r[...])
                  + bh1_r[...])
        hh = _bdot(t, wh2_r[...]) + bh2_r[...]
        pre = h_ + hh
        mu = jnp.mean(pre, axis=1, keepdims=True)
        d = pre - mu
        var = jnp.mean(d * d, axis=1, keepdims=True)
        hn = d * lax.rsqrt(var + 1e-5) * g_r[...] + b_r[...]
        xn = x + cu_r[:, 0:3] * rc
        out_r[...] = jnp.concatenate(
            [hn, xn, jnp.zeros((BLK_N, PKW - 67), jnp.float32)], axis=1)

    return pl.pallas_call(
        body,
        grid=(N_NODES // BLK_N,),
        in_specs=[
            pl.BlockSpec((BLK_N, PKW), lambda i: (i, 0)),
            pl.BlockSpec((BLK_N, 64), lambda i: (i, 0)),
            pl.BlockSpec((BLK_N, 16), lambda i: (i, 0)),
            pl.BlockSpec((BLK_N, 16), lambda i: (i, 0)),
            _rep((64, 64)), _rep((64, 64)), _rep((1, 64)),
            _rep((64, 64)), _rep((1, 64)), _rep((1, 64)), _rep((1, 64)),
        ],
        out_specs=pl.BlockSpec((BLK_N, PKW), lambda i: (i, 0)),
        out_shape=jax.ShapeDtypeStruct((N_NODES, PKW), jnp.float32),
    )(tab, agg, cu, cnt, wh1h, wh1a, bh1, wh2, bh2, ln_g, ln_b)


def _encoder(nf, coords, w0, b0, w1, b1):
    def body(nf_r, x_r, w0_r, b0_r, w1_r, b1_r, out_r):
        t = _silu(_bdot(nf_r[...], w0_r[...]) + b0_r[...])
        h = _bdot(t, w1_r[...]) + b1_r[...]
        out_r[...] = jnp.concatenate(
            [h, x_r[...], jnp.zeros((BLK_N, PKW - 67), jnp.float32)], axis=1)

    return pl.pallas_call(
        body,
        grid=(N_NODES // BLK_N,),
        in_specs=[
            pl.BlockSpec((BLK_N, D_IN), lambda i: (i, 0)),
            pl.BlockSpec((BLK_N, 3), lambda i: (i, 0)),
            _rep((D_IN, 64)), _rep((1, 64)), _rep((64, 64)), _rep((1, 64)),
        ],
        out_specs=pl.BlockSpec((BLK_N, PKW), lambda i: (i, 0)),
        out_shape=jax.ShapeDtypeStruct((N_NODES, PKW), jnp.float32),
    )(nf, coords, w0, b0, w1, b1)


def _readout(tab, batch2, r0, br0, r1, br1, r2, br2):
    n_steps = N_NODES // BLK_N

    def body(tab_r, b_r, r0_r, br0_r, r1_r, br1_r, r2_r, br2_r, out_r,
             gh_acc, ct_acc):
        i = pl.program_id(0)

        @pl.when(i == 0)
        def _():
            gh_acc[...] = jnp.zeros((N_GRAPHS, 64), jnp.float32)
            ct_acc[...] = jnp.zeros((N_GRAPHS, 1), jnp.float32)

        h_ = tab_r[:, 0:64]
        gid = jax.lax.broadcasted_iota(jnp.int32, (BLK_N, N_GRAPHS), 1)
        z = (b_r[...] == gid).astype(jnp.float32)
        gh_acc[...] += lax.dot_general(
            z, h_, (((0,), (0,)), ((), ())),
            preferred_element_type=jnp.float32)
        ct_acc[...] += lax.dot_general(
            z, jnp.ones((BLK_N, 1), jnp.float32), (((0,), (0,)), ((), ())),
            preferred_element_type=jnp.float32)

        @pl.when(i == n_steps - 1)
        def _():
            gm = gh_acc[...] / jnp.maximum(ct_acc[...], 1.0)
            g0 = _silu(_bdot(gm, r0_r[...]) + br0_r[...])
            g1 = _silu(_bdot(g0, r1_r[...]) + br1_r[...])
            out_r[...] = _bdot(g1, r2_r[...]) + br2_r[...]

    return pl.pallas_call(
        body,
        grid=(n_steps,),
        in_specs=[
            pl.BlockSpec((BLK_N, PKW), lambda i: (i, 0)),
            pl.BlockSpec((BLK_N, 1), lambda i: (i, 0)),
            _rep((64, 64)), _rep((1, 64)), _rep((64, 32)), _rep((1, 32)),
            _rep((32, 2)), _rep((1, 2)),
        ],
        out_specs=pl.BlockSpec((N_GRAPHS, 2), lambda i: (0, 0)),
        out_shape=jax.ShapeDtypeStruct((N_GRAPHS, 2), jnp.float32),
        scratch_shapes=[
            pltpu.VMEM((N_GRAPHS, 64), jnp.float32),
            pltpu.VMEM((N_GRAPHS, 1), jnp.float32),
        ],
    )(tab, batch2, r0, br0, r1, br1, r2, br2)


def _row(b):
    return b.reshape(1, -1)


def kernel(node_feats, coords, edge_index, edge_feats, batch, params):
    f32 = jnp.float32
    src = edge_index[0].astype(jnp.int32)
    dst = edge_index[1].astype(jnp.int32)
    npad = E_PAD - N_EDGES
    zpad_i = jnp.zeros((npad,), jnp.int32)
    oor = jnp.full((npad,), OOR, jnp.int32)
    src_g = jnp.concatenate([src, zpad_i])
    dst_g = jnp.concatenate([dst, zpad_i])
    src_s = jnp.concatenate([src, oor])
    dst_s = jnp.concatenate([dst, oor])
    ef_p = jnp.concatenate(
        [edge_feats.astype(bf16), jnp.zeros((npad, D_E), bf16)], axis=0)
    batch2 = batch.astype(jnp.int32).reshape(N_NODES, 1)

    enc0, enc1 = params["enc"]
    table = _encoder(node_feats.astype(f32), coords.astype(f32),
                     enc0["W"].T, _row(enc0["b"]), enc1["W"].T,
                     _row(enc1["b"]))

    cnt = _sc_counts(dst_s)

    for p in params["layers"]:
        w1 = p["e1"]["W"]
        gs, gd = _sc_gather(table, src_g, dst_g)
        msg, wd = _edge_mlp(
            gs, gd, ef_p,
            w1[:, 0:64].T.astype(bf16), w1[:, 64:128].T.astype(bf16),
            _row(w1[:, 128]), w1[:, 129:145].T.astype(bf16),
            _row(p["e1"]["b"]),
            p["e2"]["W"].T.astype(bf16), _row(p["e2"]["b"]),
            p["x1"]["W"].T.astype(bf16), _row(p["x1"]["b"]),
            p["x2"]["W"].T.astype(bf16), _row(p["x2"]["b"]),
        )
        agg = _sc_scatter(msg, dst_s, 64)
        cu = _sc_scatter(wd, src_s, 16)
        wh1 = p["h1"]["W"]
        table = _node_update(
            table, agg, cu, cnt,
            wh1[:, 0:64].T, wh1[:, 64:128].T, _row(p["h1"]["b"]),
            p["h2"]["W"].T, _row(p["h2"]["b"]),
            _row(p["ln_g"]), _row(p["ln_b"]),
        )

    r = params["ro"]
    return _readout(table, batch2, r[0]["W"].T, _row(r[0]["b"]),
                    r[1]["W"].T, _row(r[1]["b"]), r[2]["W"].T,
                    _row(r[2]["b"]))
